# baseline (speedup 1.0000x reference)
import functools

import jax
import jax.numpy as jnp
from jax import lax
from jax.experimental import pallas as pl
from jax.experimental.pallas import tpu as pltpu
from jax.experimental.pallas import tpu_sc as plsc

NC = 2
NS = 16
NW = NC * NS
L = 16
CP = 32
CHUNK = 128


def _rsqrt_newton(d):
    yi = jnp.int32(0x5F3759DF) - (lax.bitcast_convert_type(d, jnp.int32) >> 1)
    y = lax.bitcast_convert_type(yi, jnp.float32)
    for _ in range(3):
        y = y * (1.5 - 0.5 * d * y * y)
    return y


def _make_sc_scatter(n_pad, e_pad, n_ch):
    rows_total = e_pad // CHUNK
    rows_deg = rows_total // NS
    rows_z = rows_total // NW
    nslice = n_pad // NS
    zrows = 64

    mesh = plsc.VectorSubcoreMesh(
        core_axis_name="c", subcore_axis_name="s",
        num_cores=NC, num_subcores=NS)

    def body(src2_h, dst2_h, ew2_h, xpad_h,
             zp_h, dinv_h,
             Zs, dS, Xs,
             dinv_l, dstd, ewd, srcb, dstb, ewb,
             xbufA, xbufB, valbufA, valbufB, ztile, zbuf1,
             semA, semB, semS):
        c = lax.axis_index("c")
        s = lax.axis_index("s")
        wid = s * NC + c
        zero16 = jnp.zeros((L,), jnp.float32)
        base = s * nslice

        with jax.named_scope("phA_zero"):
            @plsc.parallel_loop(0, zrows)
            def _(r):
                ztile[r, pl.ds(0, L)] = zero16
                ztile[r, pl.ds(L, L)] = zero16

            @plsc.parallel_loop(0, nslice // L)
            def _(i):
                zbuf1[pl.ds(i * L, L)] = zero16

            @plsc.parallel_loop(0, CHUNK)
            def _(r):
                valbufA[r, pl.ds(L, L)] = zero16
                valbufB[r, pl.ds(L, L)] = zero16

            for t in range(nslice // zrows):
                pltpu.sync_copy(ztile, Zs.at[pl.ds(base + t * zrows, zrows)])
            pltpu.sync_copy(zbuf1, dS.at[pl.ds(base, nslice)])

        with jax.named_scope("phX_stage"):
            pltpu.sync_copy(xpad_h.at[pl.ds(base, nslice)],
                            Xs.at[pl.ds(base, nslice)])
        plsc.subcore_barrier()

        with jax.named_scope("phB_deg"):
            pltpu.sync_copy(dst2_h.at[pl.ds(s * rows_deg, rows_deg)], dstd)
            pltpu.sync_copy(ew2_h.at[pl.ds(s * rows_deg, rows_deg)], ewd)

            DEG_W = 8

            def degstep(j, _):
                pltpu.async_copy(ewd.at[j], dS.at[dstd.at[j]], semS,
                                 add=True)

                @pl.when(j >= DEG_W)
                def _():
                    pltpu.make_async_copy(ewd.at[0], dS.at[dstd.at[0]],
                                          semS).wait()
                return 0
            lax.fori_loop(0, rows_deg, degstep, 0)

            def degdrain(j, _):
                pltpu.make_async_copy(ewd.at[0], dS.at[dstd.at[0]],
                                      semS).wait()
                return 0
            lax.fori_loop(0, DEG_W, degdrain, 0)
        plsc.subcore_barrier()

        with jax.named_scope("phC_dinv"):
            pltpu.sync_copy(dS.at[pl.ds(base, nslice)], zbuf1)

            @plsc.parallel_loop(0, nslice // L)
            def _(i):
                d = zbuf1[pl.ds(i * L, L)] + 1.0
                dinv_l[pl.ds(base + i * L, L)] = _rsqrt_newton(d)

            pltpu.sync_copy(dinv_l.at[pl.ds(base, nslice)],
                            dS.at[pl.ds(base, nslice)])

            @pl.when(c == 0)
            def _():
                pltpu.sync_copy(dinv_l.at[pl.ds(base, nslice)],
                                dinv_h.at[pl.ds(base, nslice)])
            plsc.subcore_barrier()
            pltpu.sync_copy(dS, dinv_l)

        ebase = wid * rows_z
        with jax.named_scope("phD_edges_load"):
            pltpu.sync_copy(src2_h.at[pl.ds(ebase, rows_z)], srcb)
            pltpu.sync_copy(dst2_h.at[pl.ds(ebase, rows_z)], dstb)
            pltpu.sync_copy(ew2_h.at[pl.ds(ebase, rows_z)], ewb)

        def compute_val(j, xb, vb):
            @plsc.parallel_loop(0, CHUNK // L, unroll=2)
            def _(k):
                idx16 = srcb[j, pl.ds(k * L, L)]
                s16 = ewb[j, pl.ds(k * L, L)] * plsc.load_gather(dinv_l, [idx16])
                for e16 in range(L):
                    sv = s16[e16]
                    r = k * L + e16
                    vb[r, pl.ds(0, L)] = xb[r, pl.ds(0, L)] * sv
                    vb[r, pl.ds(8, L)] = xb[r, pl.ds(8, L)] * sv

        npairs = rows_z // 2
        scope_d = jax.named_scope("phD_scatter")
        scope_d.__enter__()
        pltpu.async_copy(Xs.at[srcb.at[0]], xbufA, semA)

        def pair(p, _):
            j0 = 2 * p
            j1 = j0 + 1
            pltpu.async_copy(Xs.at[srcb.at[j1]], xbufB, semB)
            pltpu.make_async_copy(Xs.at[srcb.at[j0]], xbufA, semA).wait()

            @pl.when(p > 0)
            def _():
                pltpu.make_async_copy(valbufA, Zs.at[dstb.at[0]], semS).wait()
            compute_val(j0, xbufA, valbufA)
            pltpu.async_copy(valbufA, Zs.at[dstb.at[j0]], semS, add=True)

            @pl.when(p < npairs - 1)
            def _():
                pltpu.async_copy(Xs.at[srcb.at[j0 + 2]], xbufA, semA)
            pltpu.make_async_copy(Xs.at[srcb.at[j1]], xbufB, semB).wait()

            @pl.when(p > 0)
            def _():
                pltpu.make_async_copy(valbufB, Zs.at[dstb.at[0]], semS).wait()
            compute_val(j1, xbufB, valbufB)
            pltpu.async_copy(valbufB, Zs.at[dstb.at[j1]], semS, add=True)
            return 0
        lax.fori_loop(0, npairs, pair, 0)
        pltpu.make_async_copy(valbufA, Zs.at[dstb.at[0]], semS).wait()
        pltpu.make_async_copy(valbufB, Zs.at[dstb.at[0]], semS).wait()
        scope_d.__exit__(None, None, None)
        plsc.subcore_barrier()

        with jax.named_scope("phE_out"):
            @pl.when(c == 0)
            def _():
                def echunk(t, _):
                    rb = base + t * CHUNK
                    pltpu.sync_copy(Zs.at[pl.ds(rb, CHUNK)], valbufA)
                    pltpu.sync_copy(Xs.at[pl.ds(rb, CHUNK)], xbufA)

                    @plsc.parallel_loop(0, CHUNK // L, unroll=2)
                    def _(k):
                        dv16 = dinv_l[pl.ds(rb + k * L, L)]
                        for e16 in range(L):
                            sv = dv16[e16]
                            r = k * L + e16
                            a0 = valbufA[r, pl.ds(0, L)]
                            a1 = valbufA[r, pl.ds(8, L)]
                            valbufA[r, pl.ds(0, L)] = (
                                a0 + xbufA[r, pl.ds(0, L)] * sv)
                            valbufA[r, pl.ds(8, L)] = (
                                a1 + xbufA[r, pl.ds(8, L)] * sv)
                    pltpu.sync_copy(valbufA, zp_h.at[c, pl.ds(rb, CHUNK)])
                    return 0
                lax.fori_loop(0, nslice // CHUNK, echunk, 0)

            @pl.when(c == 1)
            def _():
                pltpu.sync_copy(Zs.at[pl.ds(base, nslice)],
                                zp_h.at[c, pl.ds(base, nslice)])

    return pl.kernel(
        body,
        out_type=[
            jax.ShapeDtypeStruct((NC, n_pad, CP), jnp.float32),
            jax.ShapeDtypeStruct((n_pad,), jnp.float32),
        ],
        mesh=mesh,
        compiler_params=pltpu.CompilerParams(
            needs_layout_passes=False, use_tc_tiling_on_sc=False),
        scratch_types=[
            pltpu.VMEM_SHARED((n_pad, CP), jnp.float32),
            pltpu.VMEM_SHARED((n_pad,), jnp.float32),
            pltpu.VMEM_SHARED((n_pad, 24), jnp.float32),
            pltpu.VMEM((n_pad,), jnp.float32),
            pltpu.VMEM((rows_deg, CHUNK), jnp.int32),
            pltpu.VMEM((rows_deg, CHUNK), jnp.float32),
            pltpu.VMEM((rows_z, CHUNK), jnp.int32),
            pltpu.VMEM((rows_z, CHUNK), jnp.int32),
            pltpu.VMEM((rows_z, CHUNK), jnp.float32),
            pltpu.VMEM((CHUNK, 24), jnp.float32),
            pltpu.VMEM((CHUNK, 24), jnp.float32),
            pltpu.VMEM((CHUNK, CP), jnp.float32),
            pltpu.VMEM((CHUNK, CP), jnp.float32),
            pltpu.VMEM((64, CP), jnp.float32),
            pltpu.VMEM((n_pad // NS,), jnp.float32),
            pltpu.SemaphoreType.DMA,
            pltpu.SemaphoreType.DMA,
            pltpu.SemaphoreType.DMA,
        ],
        name="gcn_edge_scatter_sc",
    )


def _tc_body(zp, dvr, wb1, wb2, cbr, vcr, br, out):
    zsum = zp[0] + zp[1]
    y1 = zsum.astype(jnp.bfloat16)
    y2 = (zsum - y1.astype(jnp.float32)).astype(jnp.bfloat16)
    dn = (((1,), (1,)), ((), ()))
    m1 = (lax.dot_general(wb1[:, :], y1, dn,
                          preferred_element_type=jnp.float32)
          + lax.dot_general(wb1[:, :], y2, dn,
                            preferred_element_type=jnp.float32)
          + lax.dot_general(wb2[:, :], y1, dn,
                            preferred_element_type=jnp.float32))
    a = dvr[:, :] * m1
    r = jnp.maximum(a + cbr[:, :], 0.0) * vcr[:, :]
    h = wb1.shape[0] // 2
    out[0, :, :] = jnp.sum(r[0:h], axis=0, keepdims=True) + br[0, 0]
    out[1, :, :] = jnp.sum(r[h:], axis=0, keepdims=True) + br[0, 0]


def kernel(x_seq, edge_index, edge_weight, W_gcn, b_gcn, W_conv, b_conv,
           W_ro, b_ro):
    B, T, n = x_seq.shape
    TH = W_conv.shape[0]
    C = B * T
    e = edge_index.shape[1]

    n_pad = ((n + NS * L * L - 1) // (NS * L * L)) * (NS * L * L)
    e_pad = ((e + NW * CHUNK - 1) // (NW * CHUNK)) * (NW * CHUNK)

    xpad = jnp.pad(x_seq.reshape(C, n).T, ((0, n_pad - n), (0, 0)))
    src2 = jnp.pad(edge_index[0], (0, e_pad - e)).reshape(-1, CHUNK)
    dst2 = jnp.pad(edge_index[1], (0, e_pad - e)).reshape(-1, CHUNK)
    ew2 = jnp.pad(edge_weight, (0, e_pad - e)).reshape(-1, CHUNK)

    zp, dinv = _make_sc_scatter(n_pad, e_pad, C)(src2, dst2, ew2, xpad)

    dep = jnp.minimum(dinv[0], 0.0)

    wk = jnp.einsum("ogk,g->ok", W_conv, W_gcn[0]) + dep
    bg = jnp.einsum("ogk,g->ok", W_conv, b_gcn)
    tt = jnp.arange(T)[None, :]
    cb = (b_conv[:, None] + bg[:, 1:2] + bg[:, 0:1] * (tt >= 1)
          + bg[:, 2:3] * (tt <= T - 2))
    b_grid = jnp.arange(B)[:, None, None, None]
    t_grid = jnp.arange(T)[None, None, :, None]
    c_grid = jnp.arange(CP)[None, None, None, :]
    wfull = jnp.zeros((B, TH, T, CP), jnp.float32)
    for k in range(3):
        tm = t_grid + (k - 1)
        mask = (tm >= 0) & (tm < T) & (c_grid == b_grid * T + tm)
        wfull = wfull + jnp.where(mask, wk[:, k][None, :, None, None], 0.0)
    w_all = wfull.reshape(B * TH * T, CP)
    cb2 = jnp.concatenate([cb.reshape(-1), cb.reshape(-1)])[:, None]
    vcol = jnp.broadcast_to(W_ro[:, 0:1] / T, (TH, T)).reshape(-1)
    vcol2 = jnp.concatenate([vcol, vcol])[:, None]
    br = b_ro.reshape(1, 1)
    wb1 = w_all.astype(jnp.bfloat16)
    wb2 = (w_all - wb1.astype(jnp.float32)).astype(jnp.bfloat16)

    NB = 2048
    grid = n_pad // NB
    out_full = pl.pallas_call(
        _tc_body,
        grid=(grid,),
        in_specs=[
            pl.BlockSpec((NC, NB, CP), lambda i: (0, i, 0)),
            pl.BlockSpec((1, NB), lambda i: (0, i)),
            pl.BlockSpec((B * TH * T, CP), lambda i: (0, 0)),
            pl.BlockSpec((B * TH * T, CP), lambda i: (0, 0)),
            pl.BlockSpec((B * TH * T, 1), lambda i: (0, 0)),
            pl.BlockSpec((B * TH * T, 1), lambda i: (0, 0)),
            pl.BlockSpec((1, 1), lambda i: (0, 0)),
        ],
        out_specs=pl.BlockSpec((B, 1, NB), lambda i: (0, 0, i)),
        out_shape=jax.ShapeDtypeStruct((B, 1, n_pad), jnp.float32),
        name="temporal_conv_readout_tc",
    )(zp, dinv.reshape(1, n_pad), wb1, wb2, cb2, vcol2, br)

    return out_full[:, :, :n]


# speedup vs baseline: 1.0301x; 1.0301x over previous
import functools

import jax
import jax.numpy as jnp
from jax import lax
from jax.experimental import pallas as pl
from jax.experimental.pallas import tpu as pltpu
from jax.experimental.pallas import tpu_sc as plsc

NC = 2
NS = 16
NW = NC * NS
L = 16
CP = 32
CHUNK = 128


def _rsqrt_newton(d):
    yi = jnp.int32(0x5F3759DF) - (lax.bitcast_convert_type(d, jnp.int32) >> 1)
    y = lax.bitcast_convert_type(yi, jnp.float32)
    for _ in range(3):
        y = y * (1.5 - 0.5 * d * y * y)
    return y


def _make_sc_scatter(n_pad, e_pad, n_ch):
    rows_total = e_pad // CHUNK
    rows_deg = rows_total // NS
    rows_z = rows_total // NW
    nslice = n_pad // NS
    zrows = 64

    mesh = plsc.VectorSubcoreMesh(
        core_axis_name="c", subcore_axis_name="s",
        num_cores=NC, num_subcores=NS)

    def body(src2_h, dst2_h, ew2_h, xpad_h,
             zp_h, dinv_h,
             Zs, dS, Xs,
             dinv_l, dstd, ewd, srcb, dstb, ewb,
             xbufA, xbufB, valbufA, valbufB, ztile, zbuf1,
             semA, semB, semS):
        c = lax.axis_index("c")
        s = lax.axis_index("s")
        wid = s * NC + c
        zero16 = jnp.zeros((L,), jnp.float32)
        base = s * nslice

        with jax.named_scope("phA_zero"):
            @plsc.parallel_loop(0, zrows)
            def _(r):
                ztile[r, pl.ds(0, L)] = zero16
                ztile[r, pl.ds(L, L)] = zero16

            @plsc.parallel_loop(0, nslice // L)
            def _(i):
                zbuf1[pl.ds(i * L, L)] = zero16

            @plsc.parallel_loop(0, CHUNK)
            def _(r):
                valbufA[r, pl.ds(L, L)] = zero16
                valbufB[r, pl.ds(L, L)] = zero16

            for t in range(nslice // zrows):
                pltpu.sync_copy(ztile, Zs.at[pl.ds(base + t * zrows, zrows)])
            pltpu.sync_copy(zbuf1, dS.at[pl.ds(base, nslice)])

        with jax.named_scope("phX_stage"):
            pltpu.sync_copy(xpad_h.at[pl.ds(base, nslice)],
                            Xs.at[pl.ds(base, nslice)])
        plsc.subcore_barrier()

        with jax.named_scope("phB_deg"):
            pltpu.sync_copy(dst2_h.at[pl.ds(s * rows_deg, rows_deg)], dstd)
            pltpu.sync_copy(ew2_h.at[pl.ds(s * rows_deg, rows_deg)], ewd)

            DEG_W = 8

            def degstep(j, _):
                pltpu.async_copy(ewd.at[j], dS.at[dstd.at[j]], semS,
                                 add=True)

                @pl.when(j >= DEG_W)
                def _():
                    pltpu.make_async_copy(ewd.at[0], dS.at[dstd.at[0]],
                                          semS).wait()
                return 0
            lax.fori_loop(0, rows_deg, degstep, 0)

            def degdrain(j, _):
                pltpu.make_async_copy(ewd.at[0], dS.at[dstd.at[0]],
                                      semS).wait()
                return 0
            lax.fori_loop(0, DEG_W, degdrain, 0)
        plsc.subcore_barrier()

        with jax.named_scope("phC_dinv"):
            pltpu.sync_copy(dS.at[pl.ds(base, nslice)], zbuf1)

            @plsc.parallel_loop(0, nslice // L)
            def _(i):
                d = zbuf1[pl.ds(i * L, L)] + 1.0
                dinv_l[pl.ds(base + i * L, L)] = _rsqrt_newton(d)

            pltpu.sync_copy(dinv_l.at[pl.ds(base, nslice)],
                            dS.at[pl.ds(base, nslice)])

            @pl.when(c == 0)
            def _():
                pltpu.sync_copy(dinv_l.at[pl.ds(base, nslice)],
                                dinv_h.at[pl.ds(base, nslice)])
            plsc.subcore_barrier()
            pltpu.sync_copy(dS, dinv_l)

        ebase = wid * rows_z
        with jax.named_scope("phD_edges_load"):
            pltpu.sync_copy(src2_h.at[pl.ds(ebase, rows_z)], srcb)
            pltpu.sync_copy(dst2_h.at[pl.ds(ebase, rows_z)], dstb)
            pltpu.sync_copy(ew2_h.at[pl.ds(ebase, rows_z)], ewb)

        def compute_val(j, xb, vb):
            @plsc.parallel_loop(0, CHUNK // L, unroll=2)
            def _(k):
                idx16 = srcb[j, pl.ds(k * L, L)]
                s16 = ewb[j, pl.ds(k * L, L)] * plsc.load_gather(dinv_l, [idx16])
                for e16 in range(L):
                    sv = s16[e16]
                    r = k * L + e16
                    vb[r, pl.ds(0, L)] = xb[r, pl.ds(0, L)] * sv
                    vb[r, pl.ds(8, L)] = xb[r, pl.ds(8, L)] * sv

        npairs = rows_z // 2
        scope_d = jax.named_scope("phD_scatter")
        scope_d.__enter__()
        pltpu.async_copy(Xs.at[srcb.at[0]], xbufA, semA)

        def pair(p, _):
            j0 = 2 * p
            j1 = j0 + 1
            pltpu.async_copy(Xs.at[srcb.at[j1]], xbufB, semB)
            pltpu.make_async_copy(Xs.at[srcb.at[j0]], xbufA, semA).wait()

            @pl.when(p > 0)
            def _():
                pltpu.make_async_copy(valbufA, Zs.at[dstb.at[0]], semS).wait()
            compute_val(j0, xbufA, valbufA)
            pltpu.async_copy(valbufA, Zs.at[dstb.at[j0]], semS, add=True)

            @pl.when(p < npairs - 1)
            def _():
                pltpu.async_copy(Xs.at[srcb.at[j0 + 2]], xbufA, semA)
            pltpu.make_async_copy(Xs.at[srcb.at[j1]], xbufB, semB).wait()

            @pl.when(p > 0)
            def _():
                pltpu.make_async_copy(valbufB, Zs.at[dstb.at[0]], semS).wait()
            compute_val(j1, xbufB, valbufB)
            pltpu.async_copy(valbufB, Zs.at[dstb.at[j1]], semS, add=True)
            return 0
        lax.fori_loop(0, npairs, pair, 0)
        pltpu.make_async_copy(valbufA, Zs.at[dstb.at[0]], semS).wait()
        pltpu.make_async_copy(valbufB, Zs.at[dstb.at[0]], semS).wait()
        scope_d.__exit__(None, None, None)
        plsc.subcore_barrier()

        with jax.named_scope("phE_out"):
            @pl.when(c == 0)
            def _():
                def echunk(t, _):
                    rb = base + t * CHUNK
                    pltpu.sync_copy(Zs.at[pl.ds(rb, CHUNK)], valbufA)
                    pltpu.sync_copy(Xs.at[pl.ds(rb, CHUNK)], xbufA)

                    @plsc.parallel_loop(0, CHUNK // L, unroll=2)
                    def _(k):
                        dv16 = dinv_l[pl.ds(rb + k * L, L)]
                        for e16 in range(L):
                            sv = dv16[e16]
                            r = k * L + e16
                            a0 = valbufA[r, pl.ds(0, L)]
                            a1 = valbufA[r, pl.ds(8, L)]
                            valbufA[r, pl.ds(0, L)] = (
                                a0 + xbufA[r, pl.ds(0, L)] * sv)
                            valbufA[r, pl.ds(8, L)] = (
                                a1 + xbufA[r, pl.ds(8, L)] * sv)
                    pltpu.sync_copy(valbufA, zp_h.at[c, pl.ds(rb, CHUNK)])
                    return 0
                lax.fori_loop(0, nslice // CHUNK, echunk, 0)

            @pl.when(c == 1)
            def _():
                pltpu.sync_copy(Zs.at[pl.ds(base, nslice)],
                                zp_h.at[c, pl.ds(base, nslice)])

    return pl.kernel(
        body,
        out_type=[
            jax.ShapeDtypeStruct((NC, n_pad, CP), jnp.float32),
            jax.ShapeDtypeStruct((n_pad,), jnp.float32),
        ],
        mesh=mesh,
        compiler_params=pltpu.CompilerParams(
            needs_layout_passes=False, use_tc_tiling_on_sc=False),
        scratch_types=[
            pltpu.VMEM_SHARED((n_pad, CP), jnp.float32),
            pltpu.VMEM_SHARED((n_pad,), jnp.float32),
            pltpu.VMEM_SHARED((n_pad, 24), jnp.float32),
            pltpu.VMEM((n_pad,), jnp.float32),
            pltpu.VMEM((rows_deg, CHUNK), jnp.int32),
            pltpu.VMEM((rows_deg, CHUNK), jnp.float32),
            pltpu.VMEM((rows_z, CHUNK), jnp.int32),
            pltpu.VMEM((rows_z, CHUNK), jnp.int32),
            pltpu.VMEM((rows_z, CHUNK), jnp.float32),
            pltpu.VMEM((CHUNK, 24), jnp.float32),
            pltpu.VMEM((CHUNK, 24), jnp.float32),
            pltpu.VMEM((CHUNK, CP), jnp.float32),
            pltpu.VMEM((CHUNK, CP), jnp.float32),
            pltpu.VMEM((64, CP), jnp.float32),
            pltpu.VMEM((n_pad // NS,), jnp.float32),
            pltpu.SemaphoreType.DMA,
            pltpu.SemaphoreType.DMA,
            pltpu.SemaphoreType.DMA,
        ],
        name="gcn_edge_scatter_sc",
    )


def _tc_body(zp, dvr, wb1, wb2, cbr, vcr, br, out):
    zsum = zp[0] + zp[1]
    y1 = zsum.astype(jnp.bfloat16)
    y2 = (zsum - y1.astype(jnp.float32)).astype(jnp.bfloat16)
    dn = (((1,), (1,)), ((), ()))
    m1 = (lax.dot_general(wb1[:, :], y1, dn,
                          preferred_element_type=jnp.float32)
          + lax.dot_general(wb1[:, :], y2, dn,
                            preferred_element_type=jnp.float32)
          + lax.dot_general(wb2[:, :], y1, dn,
                            preferred_element_type=jnp.float32))
    a = dvr[:, :] * m1
    r = jnp.maximum(a + cbr[:, :], 0.0) * vcr[:, :]
    h = wb1.shape[0] // 2
    out[0, :, :] = jnp.sum(r[0:h], axis=0, keepdims=True) + br[0, 0]
    out[1, :, :] = jnp.sum(r[h:], axis=0, keepdims=True) + br[0, 0]


def kernel(x_seq, edge_index, edge_weight, W_gcn, b_gcn, W_conv, b_conv,
           W_ro, b_ro):
    B, T, n = x_seq.shape
    TH = W_conv.shape[0]
    C = B * T
    e = edge_index.shape[1]

    n_pad = ((n + NS * L * L - 1) // (NS * L * L)) * (NS * L * L)
    e_pad = ((e + NW * CHUNK - 1) // (NW * CHUNK)) * (NW * CHUNK)

    xpad = jnp.pad(x_seq.reshape(C, n).T, ((0, n_pad - n), (0, 0)))
    src2 = jnp.pad(edge_index[0], (0, e_pad - e)).reshape(-1, CHUNK)
    dst2 = jnp.pad(edge_index[1], (0, e_pad - e)).reshape(-1, CHUNK)
    ew2 = jnp.pad(edge_weight, (0, e_pad - e)).reshape(-1, CHUNK)

    wk = jnp.einsum("ogk,g->ok", W_conv, W_gcn[0])
    bg = jnp.einsum("ogk,g->ok", W_conv, b_gcn)
    tt = jnp.arange(T)[None, :]
    cb = (b_conv[:, None] + bg[:, 1:2] + bg[:, 0:1] * (tt >= 1)
          + bg[:, 2:3] * (tt <= T - 2))
    b_grid = jnp.arange(B)[:, None, None, None]
    t_grid = jnp.arange(T)[None, None, :, None]
    c_grid = jnp.arange(CP)[None, None, None, :]
    wfull = jnp.zeros((B, TH, T, CP), jnp.float32)
    for k in range(3):
        tm = t_grid + (k - 1)
        mask = (tm >= 0) & (tm < T) & (c_grid == b_grid * T + tm)
        wfull = wfull + jnp.where(mask, wk[:, k][None, :, None, None], 0.0)
    w_all = wfull.reshape(B * TH * T, CP)
    cb2 = jnp.concatenate([cb.reshape(-1), cb.reshape(-1)])[:, None]
    vcol = jnp.broadcast_to(W_ro[:, 0:1] / T, (TH, T)).reshape(-1)
    vcol2 = jnp.concatenate([vcol, vcol])[:, None]
    br = b_ro.reshape(1, 1)
    wb1 = w_all.astype(jnp.bfloat16)
    wb2 = (w_all - wb1.astype(jnp.float32)).astype(jnp.bfloat16)

    zp, dinv = _make_sc_scatter(n_pad, e_pad, C)(src2, dst2, ew2, xpad)

    NB = 2048
    grid = n_pad // NB
    out_full = pl.pallas_call(
        _tc_body,
        grid=(grid,),
        in_specs=[
            pl.BlockSpec((NC, NB, CP), lambda i: (0, i, 0)),
            pl.BlockSpec((1, NB), lambda i: (0, i)),
            pl.BlockSpec((B * TH * T, CP), lambda i: (0, 0)),
            pl.BlockSpec((B * TH * T, CP), lambda i: (0, 0)),
            pl.BlockSpec((B * TH * T, 1), lambda i: (0, 0)),
            pl.BlockSpec((B * TH * T, 1), lambda i: (0, 0)),
            pl.BlockSpec((1, 1), lambda i: (0, 0)),
        ],
        out_specs=pl.BlockSpec((B, 1, NB), lambda i: (0, 0, i)),
        out_shape=jax.ShapeDtypeStruct((B, 1, n_pad), jnp.float32),
        name="temporal_conv_readout_tc",
    )(zp, dinv.reshape(1, n_pad), wb1, wb2, cb2, vcol2, br)

    return out_full[:, :, :n]
